# baseline (speedup 1.0000x reference)
import functools

import jax
import jax.numpy as jnp
from jax import lax
from jax.experimental import pallas as pl
from jax.experimental.pallas import tpu as pltpu
from jax.experimental.pallas import tpu_sc as plsc

USER_SIZE = 100000
ITEM_SIZE = 100000
EMBED = 64
B = 4096
A = 0.2

_INFO = plsc.get_sparse_core_info()
NC = _INFO.num_cores
NS = _INFO.num_subcores
NW = NC * NS
CHUNK = B // NW
GRP = CHUNK // 16
FLAT = EMBED * CHUNK


def _build_gather():
    mesh = plsc.VectorSubcoreMesh(core_axis_name="c", subcore_axis_name="s")

    @functools.partial(
        pl.kernel,
        mesh=mesh,
        out_type=jax.ShapeDtypeStruct((7, NW, FLAT), jnp.float32),
        scratch_types=[
            pltpu.VMEM((CHUNK,), jnp.int32),
            pltpu.VMEM((FLAT,), jnp.int32),
            pltpu.VMEM((FLAT,), jnp.float32),
            pltpu.SemaphoreType.DMA,
        ],
    )
    def gather_kernel(wu_hbm, wi_hbm, u_hbm, i_hbm, p1_hbm, p2_hbm, p3_hbm,
                      p4_hbm, p5_hbm, out_hbm, idx_v, flat_v, vals_v, sem):
        wid = lax.axis_index("s") * NC + lax.axis_index("c")
        base = wid * CHUNK
        streams = [(u_hbm, wu_hbm), (i_hbm, wi_hbm), (p1_hbm, wi_hbm),
                   (p2_hbm, wi_hbm), (p3_hbm, wi_hbm), (p4_hbm, wi_hbm),
                   (p5_hbm, wi_hbm)]
        for v, (ih, tbl) in enumerate(streams):
            pltpu.sync_copy(ih.at[pl.ds(base, CHUNK)], idx_v)
            cols = tuple(idx_v[pl.ds(g * 16, 16)] for g in range(GRP))

            def body(r, cols):
                roff = r * USER_SIZE
                for g in range(GRP):
                    flat_v[pl.ds(r * CHUNK + g * 16, 16)] = cols[g] + roff
                return cols

            lax.fori_loop(0, EMBED, body, cols, unroll=2)
            pltpu.async_copy(tbl.at[flat_v], vals_v, sem).wait()
            pltpu.sync_copy(vals_v, out_hbm.at[v].at[wid])

    return gather_kernel


_gather = _build_gather()


def _dense_kernel(g_ref, bu_ref, bi_ref, w1_ref, b1_ref, w2_ref, b2_ref,
                  w3_ref, b3_ref, out_ref):
    eu = jnp.maximum(g_ref[0, 0] + bu_ref[...], 0.0)
    ei = jnp.maximum(g_ref[1, 0] + bi_ref[...], 0.0)
    es = [jnp.maximum(g_ref[k, 0] + bi_ref[...], 0.0) for k in range(2, 7)]
    xs = [jnp.exp(jnp.sum(ei * e, axis=0, keepdims=True)) for e in es]
    summ = xs[0] + xs[1] + xs[2] + xs[3] + xs[4]
    pum = sum(x * e for x, e in zip(xs, es)) / summ
    pu = eu + A * pum
    x1 = jnp.concatenate([pu, ei], axis=0)
    h = jnp.maximum(
        jnp.dot(w1_ref[...], x1, preferred_element_type=jnp.float32)
        + b1_ref[...], 0.0)
    h = jnp.maximum(
        jnp.dot(w2_ref[...], h, preferred_element_type=jnp.float32)
        + b2_ref[...], 0.0)
    o = jnp.dot(w3_ref[...], h, preferred_element_type=jnp.float32) + b3_ref[...]
    out_ref[...] = 1.0 / (1.0 + jnp.exp(-o))


def _fixed(shape):
    return pl.BlockSpec(shape, lambda w: tuple(0 for _ in shape))


_dense = pl.pallas_call(
    _dense_kernel,
    grid=(NW,),
    in_specs=[
        pl.BlockSpec((7, 1, EMBED, CHUNK), lambda w: (0, w, 0, 0)),
        _fixed((EMBED, 1)),
        _fixed((EMBED, 1)),
        _fixed((2 * EMBED, 2 * EMBED)),
        _fixed((2 * EMBED, 1)),
        _fixed((EMBED, 2 * EMBED)),
        _fixed((EMBED, 1)),
        _fixed((1, EMBED)),
        _fixed((1, 1)),
    ],
    out_specs=pl.BlockSpec((1, CHUNK), lambda w: (0, w)),
    out_shape=jax.ShapeDtypeStruct((1, B), jnp.float32),
)


def kernel(Wu, bu, Wi, bi, W1, b1, W2, b2, W3, b3,
           user, item, pre1, pre2, pre3, pre4, pre5):
    i32 = jnp.int32
    gath = _gather(Wu.reshape(-1), Wi.reshape(-1),
                   user.astype(i32), item.astype(i32), pre1.astype(i32),
                   pre2.astype(i32), pre3.astype(i32), pre4.astype(i32),
                   pre5.astype(i32))
    g4 = gath.reshape(7, NW, EMBED, CHUNK)
    pred = _dense(g4, bu.reshape(EMBED, 1), bi.reshape(EMBED, 1),
                  W1, b1.reshape(2 * EMBED, 1), W2, b2.reshape(EMBED, 1),
                  W3, b3.reshape(1, 1))
    return pred.reshape(-1)


# speedup vs baseline: 1.2083x; 1.2083x over previous
import functools

import jax
import jax.numpy as jnp
from jax import lax
from jax.experimental import pallas as pl
from jax.experimental.pallas import tpu as pltpu
from jax.experimental.pallas import tpu_sc as plsc

USER_SIZE = 100000
ITEM_SIZE = 100000
EMBED = 64
B = 4096
A = 0.2

_INFO = plsc.get_sparse_core_info()
NC = _INFO.num_cores
NS = _INFO.num_subcores
NW = NC * NS
CHUNK = B // NW


def _build_gather():
    mesh = plsc.VectorSubcoreMesh(core_axis_name="c", subcore_axis_name="s")

    @functools.partial(
        pl.kernel,
        mesh=mesh,
        compiler_params=pltpu.CompilerParams(use_tc_tiling_on_sc=False),
        out_type=jax.ShapeDtypeStruct((7, B, EMBED), jnp.float32),
        scratch_types=[
            pltpu.VMEM((7, CHUNK), jnp.int32),
            pltpu.VMEM((CHUNK, EMBED), jnp.float32),
            pltpu.VMEM((CHUNK, EMBED), jnp.float32),
            pltpu.SemaphoreType.DMA,
            pltpu.SemaphoreType.DMA,
            pltpu.SemaphoreType.DMA,
            pltpu.SemaphoreType.DMA,
            pltpu.SemaphoreType.DMA,
        ],
    )
    def gather_kernel(wut_hbm, wit_hbm, u_hbm, i_hbm, p1_hbm, p2_hbm, p3_hbm,
                      p4_hbm, p5_hbm, out_hbm, idx_all, vals0, vals1,
                      isem, gsem0, gsem1, wsem0, wsem1):
        wid = lax.axis_index("s") * NC + lax.axis_index("c")
        base = wid * CHUNK
        idx_hbms = [u_hbm, i_hbm, p1_hbm, p2_hbm, p3_hbm, p4_hbm, p5_hbm]
        hs = [pltpu.async_copy(ih.at[pl.ds(base, CHUNK)], idx_all.at[v], isem)
              for v, ih in enumerate(idx_hbms)]
        for h in hs:
            h.wait()
        vals = [vals0, vals1]
        gsems = [gsem0, gsem1]
        wsems = [wsem0, wsem1]
        g = [None] * 7
        w = [None] * 7
        for v in range(7):
            tbl = wut_hbm if v == 0 else wit_hbm
            if v >= 2:
                w[v - 2].wait()
            g[v] = pltpu.async_copy(tbl.at[idx_all.at[v]], vals[v % 2],
                                    gsems[v % 2])
            if v >= 1:
                g[v - 1].wait()
                w[v - 1] = pltpu.async_copy(
                    vals[(v - 1) % 2], out_hbm.at[v - 1].at[pl.ds(base, CHUNK)],
                    wsems[(v - 1) % 2])
        g[6].wait()
        w[6] = pltpu.async_copy(vals[0], out_hbm.at[6].at[pl.ds(base, CHUNK)],
                                wsems[0])
        w[5].wait()
        w[6].wait()

    return gather_kernel


_gather = _build_gather()


def _dense_kernel(g_ref, bu_ref, bi_ref, w1t_ref, b1_ref, w2t_ref, b2_ref,
                  w3t_ref, b3_ref, out_ref):
    eu = jnp.maximum(g_ref[0] + bu_ref[...], 0.0)
    ei = jnp.maximum(g_ref[1] + bi_ref[...], 0.0)
    es = [jnp.maximum(g_ref[k] + bi_ref[...], 0.0) for k in range(2, 7)]
    xs = [jnp.exp(jnp.sum(ei * e, axis=1, keepdims=True)) for e in es]
    summ = xs[0] + xs[1] + xs[2] + xs[3] + xs[4]
    pum = sum(x * e for x, e in zip(xs, es)) / summ
    pu = eu + A * pum
    x1 = jnp.concatenate([pu, ei], axis=1)
    h = jnp.maximum(
        jnp.dot(x1, w1t_ref[...], preferred_element_type=jnp.float32)
        + b1_ref[...], 0.0)
    h = jnp.maximum(
        jnp.dot(h, w2t_ref[...], preferred_element_type=jnp.float32)
        + b2_ref[...], 0.0)
    o = jnp.dot(h, w3t_ref[...], preferred_element_type=jnp.float32) + b3_ref[...]
    out_ref[...] = 1.0 / (1.0 + jnp.exp(-o))


_dense = pl.pallas_call(
    _dense_kernel,
    out_shape=jax.ShapeDtypeStruct((B, 1), jnp.float32),
)


def kernel(Wu, bu, Wi, bi, W1, b1, W2, b2, W3, b3,
           user, item, pre1, pre2, pre3, pre4, pre5):
    i32 = jnp.int32
    gath = _gather(Wu.T, Wi.T,
                   user.astype(i32), item.astype(i32), pre1.astype(i32),
                   pre2.astype(i32), pre3.astype(i32), pre4.astype(i32),
                   pre5.astype(i32))
    pred = _dense(gath, bu.reshape(1, EMBED), bi.reshape(1, EMBED),
                  W1.T, b1.reshape(1, 2 * EMBED), W2.T, b2.reshape(1, EMBED),
                  W3.T, b3.reshape(1, 1))
    return pred.reshape(-1)


# speedup vs baseline: 1.4533x; 1.2028x over previous
import functools

import jax
import jax.numpy as jnp
from jax import lax
from jax.experimental import pallas as pl
from jax.experimental.pallas import tpu as pltpu
from jax.experimental.pallas import tpu_sc as plsc

USER_SIZE = 100000
ITEM_SIZE = 100000
EMBED = 64
EPAD = 128
B = 4096
A = 0.2

_INFO = plsc.get_sparse_core_info()
NC = _INFO.num_cores
NS = _INFO.num_subcores
NW = NC * NS
CHUNK = B // NW

_TCOLS = 1024
_TSTEPS = -(-USER_SIZE // _TCOLS)


def _transpose_kernel(wu_ref, wi_ref, wut_ref, wit_ref):
    z = jnp.zeros((_TCOLS, EMBED), jnp.float32)
    wut_ref[...] = jnp.concatenate([jnp.swapaxes(wu_ref[...], 0, 1), z], axis=1)
    wit_ref[...] = jnp.concatenate([jnp.swapaxes(wi_ref[...], 0, 1), z], axis=1)


_transpose = pl.pallas_call(
    _transpose_kernel,
    grid=(_TSTEPS,),
    in_specs=[
        pl.BlockSpec((EMBED, _TCOLS), lambda w: (0, w)),
        pl.BlockSpec((EMBED, _TCOLS), lambda w: (0, w)),
    ],
    out_specs=[
        pl.BlockSpec((_TCOLS, EPAD), lambda w: (w, 0)),
        pl.BlockSpec((_TCOLS, EPAD), lambda w: (w, 0)),
    ],
    out_shape=[
        jax.ShapeDtypeStruct((USER_SIZE, EPAD), jnp.float32),
        jax.ShapeDtypeStruct((ITEM_SIZE, EPAD), jnp.float32),
    ],
)


def _build_gather():
    mesh = plsc.VectorSubcoreMesh(core_axis_name="c", subcore_axis_name="s")

    @functools.partial(
        pl.kernel,
        mesh=mesh,
        compiler_params=pltpu.CompilerParams(use_tc_tiling_on_sc=True),
        out_type=jax.ShapeDtypeStruct((7, B, EPAD), jnp.float32),
        scratch_types=[
            pltpu.VMEM((7, CHUNK), jnp.int32),
            pltpu.VMEM((CHUNK, EPAD), jnp.float32),
            pltpu.VMEM((CHUNK, EPAD), jnp.float32),
            pltpu.SemaphoreType.DMA,
            pltpu.SemaphoreType.DMA,
            pltpu.SemaphoreType.DMA,
            pltpu.SemaphoreType.DMA,
            pltpu.SemaphoreType.DMA,
        ],
    )
    def gather_kernel(wut_hbm, wit_hbm, u_hbm, i_hbm, p1_hbm, p2_hbm, p3_hbm,
                      p4_hbm, p5_hbm, out_hbm, idx_all, vals0, vals1,
                      isem, gsem0, gsem1, wsem0, wsem1):
        wid = lax.axis_index("s") * NC + lax.axis_index("c")
        base = wid * CHUNK
        idx_hbms = [u_hbm, i_hbm, p1_hbm, p2_hbm, p3_hbm, p4_hbm, p5_hbm]
        hs = [pltpu.async_copy(ih.at[pl.ds(base, CHUNK)], idx_all.at[v], isem)
              for v, ih in enumerate(idx_hbms)]
        for h in hs:
            h.wait()
        vals = [vals0, vals1]
        gsems = [gsem0, gsem1]
        wsems = [wsem0, wsem1]
        g = [None] * 7
        w = [None] * 7
        for v in range(7):
            tbl = wut_hbm if v == 0 else wit_hbm
            if v >= 2:
                w[v - 2].wait()
            g[v] = pltpu.async_copy(tbl.at[idx_all.at[v]], vals[v % 2],
                                    gsems[v % 2])
            if v >= 1:
                g[v - 1].wait()
                w[v - 1] = pltpu.async_copy(
                    vals[(v - 1) % 2], out_hbm.at[v - 1].at[pl.ds(base, CHUNK)],
                    wsems[(v - 1) % 2])
        g[6].wait()
        w[6] = pltpu.async_copy(vals[0], out_hbm.at[6].at[pl.ds(base, CHUNK)],
                                wsems[0])
        w[5].wait()
        w[6].wait()

    return gather_kernel


_gather = _build_gather()


def _dense_kernel(g_ref, bu_ref, bi_ref, w1t_ref, b1_ref, w2t_ref, b2_ref,
                  w3t_ref, b3_ref, out_ref):
    eu = jnp.maximum(g_ref[0][:, :EMBED] + bu_ref[...], 0.0)
    ei = jnp.maximum(g_ref[1][:, :EMBED] + bi_ref[...], 0.0)
    es = [jnp.maximum(g_ref[k][:, :EMBED] + bi_ref[...], 0.0)
          for k in range(2, 7)]
    xs = [jnp.exp(jnp.sum(ei * e, axis=1, keepdims=True)) for e in es]
    summ = xs[0] + xs[1] + xs[2] + xs[3] + xs[4]
    pum = sum(x * e for x, e in zip(xs, es)) / summ
    pu = eu + A * pum
    x1 = jnp.concatenate([pu, ei], axis=1)
    h = jnp.maximum(
        jnp.dot(x1, w1t_ref[...], preferred_element_type=jnp.float32)
        + b1_ref[...], 0.0)
    h = jnp.maximum(
        jnp.dot(h, w2t_ref[...], preferred_element_type=jnp.float32)
        + b2_ref[...], 0.0)
    o = jnp.dot(h, w3t_ref[...], preferred_element_type=jnp.float32) + b3_ref[...]
    out_ref[...] = 1.0 / (1.0 + jnp.exp(-o))


_dense = pl.pallas_call(
    _dense_kernel,
    out_shape=jax.ShapeDtypeStruct((B, 1), jnp.float32),
)


def kernel(Wu, bu, Wi, bi, W1, b1, W2, b2, W3, b3,
           user, item, pre1, pre2, pre3, pre4, pre5):
    i32 = jnp.int32
    wut, wit = _transpose(Wu, Wi)
    gath = _gather(wut, wit,
                   user.astype(i32), item.astype(i32), pre1.astype(i32),
                   pre2.astype(i32), pre3.astype(i32), pre4.astype(i32),
                   pre5.astype(i32))
    pred = _dense(gath, bu.reshape(1, EMBED), bi.reshape(1, EMBED),
                  W1.T, b1.reshape(1, 2 * EMBED), W2.T, b2.reshape(1, EMBED),
                  W3.T, b3.reshape(1, 1))
    return pred.reshape(-1)
